# Initial kernel scaffold; baseline (speedup 1.0000x reference)
import jax
import jax.numpy as jnp
from jax.experimental import pallas as pl


def kernel(x, edge_index, W1, att_src1, att_dst1, b1, W2, att_src2, att_dst2, b2):
    raise NotImplementedError("write your pallas kernel here")


# speedup vs baseline: 33.3714x; 33.3714x over previous
import functools

import jax
import jax.numpy as jnp
from jax import lax
from jax.experimental import pallas as pl
from jax.experimental.pallas import tpu as pltpu
from jax.experimental.pallas import tpu_sc as plsc

F32 = jnp.float32
N = 10000
E = 320000
C = 128
NPAD = 10240
NC = 2
NS = 16
NW = NC * NS
EW = E // NW
CHUNK = 80
NCHUNKS = EW // CHUNK
RPT = NPAD // NS

_HI = lax.Precision.HIGHEST


def _node1_body(x_ref, w1_ref, asv_ref, adv_ref, xp_ref, as_ref, ad_ref):
    xb = jnp.dot(x_ref[...], w1_ref[...], preferred_element_type=F32,
                 precision=_HI)
    xp_ref[...] = xb
    as_ref[...] = jnp.dot(xb, asv_ref[...], preferred_element_type=F32,
                          precision=_HI)
    ad_ref[...] = jnp.dot(xb, adv_ref[...], preferred_element_type=F32,
                          precision=_HI)


def _node_stage1(x_pad, W1, asv, adv):
    rb = 512
    grid = NPAD // rb
    return pl.pallas_call(
        _node1_body,
        grid=(grid,),
        in_specs=[
            pl.BlockSpec((rb, C), lambda i: (i, 0)),
            pl.BlockSpec((C, C), lambda i: (0, 0)),
            pl.BlockSpec((C, 1), lambda i: (0, 0)),
            pl.BlockSpec((C, 1), lambda i: (0, 0)),
        ],
        out_specs=[
            pl.BlockSpec((rb, C), lambda i: (i, 0)),
            pl.BlockSpec((rb, 1), lambda i: (i, 0)),
            pl.BlockSpec((rb, 1), lambda i: (i, 0)),
        ],
        out_shape=[
            jax.ShapeDtypeStruct((NPAD, C), F32),
            jax.ShapeDtypeStruct((NPAD, 1), F32),
            jax.ShapeDtypeStruct((NPAD, 1), F32),
        ],
    )(x_pad, W1, asv, adv)


def _edge1_body(src_hbm, dst_hbm, asrc_hbm, adst_hbm, xp_hbm, zacc_hbm,
                zden_hbm, acc_out, den_out,
                asrc_t, adst_t, src_buf, dst_buf, w_buf, rows, acc_s, den_s,
                sem):
    c = lax.axis_index("c")
    s = lax.axis_index("s")
    w_id = c * NS + s

    pltpu.sync_copy(asrc_hbm, asrc_t)
    pltpu.sync_copy(adst_hbm, adst_t)
    pltpu.sync_copy(zacc_hbm, acc_s.at[pl.ds(s * RPT, RPT)])
    pltpu.sync_copy(zden_hbm, den_s.at[pl.ds(s * RPT, RPT)])
    plsc.subcore_barrier()

    def chunk_body(k, carry):
        base = w_id * EW + k * CHUNK
        pltpu.sync_copy(src_hbm.at[pl.ds(base, CHUNK)], src_buf)
        pltpu.sync_copy(dst_hbm.at[pl.ds(base, CHUNK)], dst_buf)
        for g in range(CHUNK // 16):
            sl = pl.ds(g * 16, 16)
            isrc = src_buf[sl]
            idst = dst_buf[sl]
            a = (plsc.load_gather(asrc_t, [isrc])
                 + plsc.load_gather(adst_t, [idst]))
            a = jnp.where(a > 0, a, 0.2 * a)
            w_buf[sl] = jnp.exp(a)
        pltpu.async_copy(xp_hbm.at[src_buf], rows, sem).wait()

        def scale_body(e, carry2):
            wv = plsc.load_gather(w_buf, [lax.broadcast(e, (16,))])
            for cc in range(C // 16):
                sl2 = pl.ds(cc * 16, 16)
                rows[e, sl2] = rows[e, sl2] * wv
            return carry2

        lax.fori_loop(0, CHUNK, scale_body, 0)
        pltpu.sync_copy(rows, acc_s.at[dst_buf], add=True)
        pltpu.sync_copy(w_buf, den_s.at[dst_buf], add=True)
        return carry

    lax.fori_loop(0, NCHUNKS, chunk_body, 0)
    plsc.subcore_barrier()
    pltpu.sync_copy(acc_s.at[pl.ds(s * RPT, RPT)],
                    acc_out.at[c, pl.ds(s * RPT, RPT)])
    pltpu.sync_copy(den_s.at[pl.ds(s * RPT, RPT)],
                    den_out.at[c, pl.ds(s * RPT, RPT)])


def _edge_stage1(src, dst, asrc, adst, xp, zacc, zden):
    mesh = plsc.VectorSubcoreMesh(core_axis_name="c", subcore_axis_name="s")
    f = pl.kernel(
        _edge1_body,
        out_type=[
            jax.ShapeDtypeStruct((NC, NPAD, C), F32),
            jax.ShapeDtypeStruct((NC, NPAD), F32),
        ],
        mesh=mesh,
        scratch_types=[
            pltpu.VMEM((NPAD,), F32),
            pltpu.VMEM((NPAD,), F32),
            pltpu.VMEM((CHUNK,), jnp.int32),
            pltpu.VMEM((CHUNK,), jnp.int32),
            pltpu.VMEM((CHUNK,), F32),
            pltpu.VMEM((CHUNK, C), F32),
            pltpu.VMEM_SHARED((NPAD, C), F32),
            pltpu.VMEM_SHARED((NPAD,), F32),
            pltpu.SemaphoreType.DMA,
        ],
        compiler_params=pltpu.CompilerParams(needs_layout_passes=False),
    )
    return f(src, dst, asrc, adst, xp, zacc, zden)


def _node2_body(accp_ref, denp_ref, b1_ref, w2_ref, as2_ref, ad2_ref,
                z_ref, zs_ref, zd_ref):
    a = accp_ref[0] + accp_ref[1]
    d = denp_ref[0] + denp_ref[1]
    h = jnp.maximum(a / (d + 1e-16) + b1_ref[...], 0.0)
    z = jnp.dot(h, w2_ref[...], preferred_element_type=F32, precision=_HI)
    z_ref[...] = z
    zs_ref[...] = z * as2_ref[0, 0]
    zd_ref[...] = z * ad2_ref[0, 0]


def _node_stage2(acc_p, den_p, b1, W2, as2, ad2):
    rb = 512
    grid = NPAD // rb
    return pl.pallas_call(
        _node2_body,
        grid=(grid,),
        in_specs=[
            pl.BlockSpec((NC, rb, C), lambda i: (0, i, 0)),
            pl.BlockSpec((NC, rb, 1), lambda i: (0, i, 0)),
            pl.BlockSpec((1, C), lambda i: (0, 0)),
            pl.BlockSpec((C, 1), lambda i: (0, 0)),
            pl.BlockSpec((1, 1), lambda i: (0, 0)),
            pl.BlockSpec((1, 1), lambda i: (0, 0)),
        ],
        out_specs=[
            pl.BlockSpec((rb, 1), lambda i: (i, 0)),
            pl.BlockSpec((rb, 1), lambda i: (i, 0)),
            pl.BlockSpec((rb, 1), lambda i: (i, 0)),
        ],
        out_shape=[
            jax.ShapeDtypeStruct((NPAD, 1), F32),
            jax.ShapeDtypeStruct((NPAD, 1), F32),
            jax.ShapeDtypeStruct((NPAD, 1), F32),
        ],
    )(acc_p, den_p, b1, W2, as2, ad2)


def _edge2_body(src_hbm, dst_hbm, zs_hbm, zd_hbm, z_hbm, zden_hbm,
                num_out, den_out,
                zs_t, zd_t, z_t, src_buf, dst_buf, w_buf, m_buf,
                num_s, den_s):
    c = lax.axis_index("c")
    s = lax.axis_index("s")
    w_id = c * NS + s

    pltpu.sync_copy(zs_hbm, zs_t)
    pltpu.sync_copy(zd_hbm, zd_t)
    pltpu.sync_copy(z_hbm, z_t)
    pltpu.sync_copy(zden_hbm, num_s.at[pl.ds(s * RPT, RPT)])
    pltpu.sync_copy(zden_hbm, den_s.at[pl.ds(s * RPT, RPT)])
    plsc.subcore_barrier()

    def chunk_body(k, carry):
        base = w_id * EW + k * CHUNK
        pltpu.sync_copy(src_hbm.at[pl.ds(base, CHUNK)], src_buf)
        pltpu.sync_copy(dst_hbm.at[pl.ds(base, CHUNK)], dst_buf)
        for g in range(CHUNK // 16):
            sl = pl.ds(g * 16, 16)
            isrc = src_buf[sl]
            idst = dst_buf[sl]
            a = (plsc.load_gather(zs_t, [isrc])
                 + plsc.load_gather(zd_t, [idst]))
            a = jnp.where(a > 0, a, 0.2 * a)
            w = jnp.exp(a)
            w_buf[sl] = w
            m_buf[sl] = w * plsc.load_gather(z_t, [isrc])
        pltpu.sync_copy(m_buf, num_s.at[dst_buf], add=True)
        pltpu.sync_copy(w_buf, den_s.at[dst_buf], add=True)
        return carry

    lax.fori_loop(0, NCHUNKS, chunk_body, 0)
    plsc.subcore_barrier()
    pltpu.sync_copy(num_s.at[pl.ds(s * RPT, RPT)],
                    num_out.at[c, pl.ds(s * RPT, RPT)])
    pltpu.sync_copy(den_s.at[pl.ds(s * RPT, RPT)],
                    den_out.at[c, pl.ds(s * RPT, RPT)])


def _edge_stage2(src, dst, zs, zd, z, zden):
    mesh = plsc.VectorSubcoreMesh(core_axis_name="c", subcore_axis_name="s")
    f = pl.kernel(
        _edge2_body,
        out_type=[
            jax.ShapeDtypeStruct((NC, NPAD), F32),
            jax.ShapeDtypeStruct((NC, NPAD), F32),
        ],
        mesh=mesh,
        scratch_types=[
            pltpu.VMEM((NPAD,), F32),
            pltpu.VMEM((NPAD,), F32),
            pltpu.VMEM((NPAD,), F32),
            pltpu.VMEM((CHUNK,), jnp.int32),
            pltpu.VMEM((CHUNK,), jnp.int32),
            pltpu.VMEM((CHUNK,), F32),
            pltpu.VMEM((CHUNK,), F32),
            pltpu.VMEM_SHARED((NPAD,), F32),
            pltpu.VMEM_SHARED((NPAD,), F32),
        ],
        compiler_params=pltpu.CompilerParams(needs_layout_passes=False),
    )
    return f(src, dst, zs, zd, z, zden)


def _final_body(nump_ref, denp_ref, b2_ref, o_ref):
    o_ref[...] = ((nump_ref[0] + nump_ref[1])
                  / (denp_ref[0] + denp_ref[1] + 1e-16)) + b2_ref[0, 0]


def _final_stage(num_p, den_p, b2):
    return pl.pallas_call(
        _final_body,
        in_specs=[
            pl.BlockSpec((NC, NPAD // C, C), lambda: (0, 0, 0)),
            pl.BlockSpec((NC, NPAD // C, C), lambda: (0, 0, 0)),
            pl.BlockSpec((1, 1), lambda: (0, 0)),
        ],
        out_specs=pl.BlockSpec((NPAD // C, C), lambda: (0, 0)),
        out_shape=jax.ShapeDtypeStruct((NPAD // C, C), F32),
    )(num_p, den_p, b2)


def kernel(x, edge_index, W1, att_src1, att_dst1, b1, W2, att_src2,
           att_dst2, b2):
    x_pad = jnp.concatenate(
        [x, jnp.zeros((NPAD - N, C), F32)], axis=0)
    src = edge_index[0].astype(jnp.int32)
    dst = edge_index[1].astype(jnp.int32)
    asv = att_src1.reshape(C, 1)
    adv = att_dst1.reshape(C, 1)

    xp, asrc, adst = _node_stage1(x_pad, W1, asv, adv)

    zacc = jnp.zeros((RPT, C), F32)
    zden = jnp.zeros((RPT,), F32)
    acc_p, den_p = _edge_stage1(src, dst, asrc.reshape(NPAD),
                                adst.reshape(NPAD), xp, zacc, zden)

    z, zs, zd = _node_stage2(acc_p, den_p.reshape(NC, NPAD, 1),
                             b1.reshape(1, C), W2,
                             att_src2.reshape(1, 1), att_dst2.reshape(1, 1))

    num_p, den2_p = _edge_stage2(src, dst, zs.reshape(NPAD),
                                 zd.reshape(NPAD), z.reshape(NPAD), zden)

    o = _final_stage(num_p.reshape(NC, NPAD // C, C),
                     den2_p.reshape(NC, NPAD // C, C), b2.reshape(1, 1))
    return o.reshape(NPAD)[:N]


# speedup vs baseline: 62.2703x; 1.8660x over previous
import functools

import jax
import jax.numpy as jnp
from jax import lax
from jax.experimental import pallas as pl
from jax.experimental.pallas import tpu as pltpu
from jax.experimental.pallas import tpu_sc as plsc

F32 = jnp.float32
N = 10000
E = 320000
C = 128
NPAD = 10240
NC = 2
NS = 16
NW = NC * NS
EW = E // NW
CHUNK = 80
NCHUNKS = EW // CHUNK
RPT = NPAD // NS
CH = C // 2

_HI = lax.Precision.HIGHEST


def _node1_body(x_ref, w1_ref, asv_ref, adv_ref, xplo_ref, xphi_ref,
                as_ref, ad_ref):
    xb = jnp.dot(x_ref[...], w1_ref[...], preferred_element_type=F32)
    xplo_ref[...] = xb[:, :CH]
    xphi_ref[...] = xb[:, CH:]
    as_ref[...] = jnp.dot(xb, asv_ref[...], preferred_element_type=F32,
                          precision=_HI)
    ad_ref[...] = jnp.dot(xb, adv_ref[...], preferred_element_type=F32,
                          precision=_HI)


def _node_stage1(x_pad, W1, asv, adv):
    rb = 512
    grid = NPAD // rb
    return pl.pallas_call(
        _node1_body,
        grid=(grid,),
        in_specs=[
            pl.BlockSpec((rb, C), lambda i: (i, 0)),
            pl.BlockSpec((C, C), lambda i: (0, 0)),
            pl.BlockSpec((C, 1), lambda i: (0, 0)),
            pl.BlockSpec((C, 1), lambda i: (0, 0)),
        ],
        out_specs=[
            pl.BlockSpec((rb, CH), lambda i: (i, 0)),
            pl.BlockSpec((rb, CH), lambda i: (i, 0)),
            pl.BlockSpec((rb, 1), lambda i: (i, 0)),
            pl.BlockSpec((rb, 1), lambda i: (i, 0)),
        ],
        out_shape=[
            jax.ShapeDtypeStruct((NPAD, CH), F32),
            jax.ShapeDtypeStruct((NPAD, CH), F32),
            jax.ShapeDtypeStruct((NPAD, 1), F32),
            jax.ShapeDtypeStruct((NPAD, 1), F32),
        ],
    )(x_pad, W1, asv, adv)


NBUF = 5


def _edge1_body(srcr_hbm, dstr_hbm, asrc_hbm, adst_hbm, xplo_hbm, xphi_hbm,
                zacc_hbm, zden_hbm, acclo_out, acchi_out, den_out,
                src_t, dst_t, w_t, asrc_t, adst_t,
                r0, r1, r2, r3, r4, acc_s, den_s,
                g0, g1, g2, g3, g4, t0, t1, t2, t3, t4, sden, szero):
    rows = [r0, r1, r2, r3, r4]
    sg = [g0, g1, g2, g3, g4]
    ss = [t0, t1, t2, t3, t4]
    c = lax.axis_index("c")
    s = lax.axis_index("s")
    w_id = c * NS + s
    sl_own = pl.ds(s * RPT, RPT)

    pltpu.async_copy(zacc_hbm, acc_s.at[sl_own], szero)
    pltpu.async_copy(zden_hbm, den_s.at[sl_own], szero)
    row0 = w_id * NCHUNKS
    pltpu.sync_copy(srcr_hbm.at[pl.ds(row0, NCHUNKS)], src_t)
    pltpu.sync_copy(dstr_hbm.at[pl.ds(row0, NCHUNKS)], dst_t)
    pltpu.sync_copy(asrc_hbm, asrc_t)
    pltpu.sync_copy(adst_hbm, adst_t)

    def wk(k, carry):
        for g in range(CHUNK // 16):
            sl = pl.ds(g * 16, 16)
            a = (plsc.load_gather(asrc_t, [src_t[k, sl]])
                 + plsc.load_gather(adst_t, [dst_t[k, sl]]))
            a = jnp.where(a > 0, a, 0.2 * a)
            w_t[k, sl] = jnp.exp(a)
        return carry

    lax.fori_loop(0, NCHUNKS, wk, 0)

    pltpu.make_async_copy(zacc_hbm, acc_s.at[sl_own], szero).wait()
    pltpu.make_async_copy(zden_hbm, den_s.at[sl_own], szero).wait()
    plsc.subcore_barrier()

    def half_pass(xp_hbm, out_ref, do_den):
        def step(j, b, b2, wait_den, wait_ss2, issue_next):
            pltpu.make_async_copy(xp_hbm.at[src_t.at[j]], rows[b],
                                  sg[b]).wait()
            jv = jnp.full((16,), j, jnp.int32)

            def scale_body(e, carry):
                wv = plsc.load_gather(w_t,
                                      [jv, jnp.full((16,), e, jnp.int32)])
                for cc in range(CH // 16):
                    sl2 = pl.ds(cc * 16, 16)
                    rows[b][e, sl2] = rows[b][e, sl2] * wv
                return carry

            lax.fori_loop(0, CHUNK, scale_body, 0, unroll=4)
            if do_den:
                if wait_den:
                    pltpu.make_async_copy(w_t.at[0], den_s.at[dst_t.at[0]],
                                          sden).wait()
                pltpu.async_copy(w_t.at[j], den_s.at[dst_t.at[j]], sden,
                                 add=True)
            pltpu.async_copy(rows[b], acc_s.at[dst_t.at[j]], ss[b], add=True)
            if issue_next:
                if wait_ss2:
                    pltpu.make_async_copy(rows[b2], acc_s.at[dst_t.at[0]],
                                          ss[b2]).wait()
                pltpu.async_copy(xp_hbm.at[src_t.at[j + 2]], rows[b2],
                                 sg[b2])

        pltpu.async_copy(xp_hbm.at[src_t.at[0]], rows[0], sg[0])
        pltpu.async_copy(xp_hbm.at[src_t.at[1]], rows[1], sg[1])
        step(0, 0, 2, False, False, True)
        step(1, 1, 3, True, False, True)
        step(2, 2, 4, True, False, True)

        def main_body(jj, carry):
            for i in range(NBUF):
                step(3 + jj * NBUF + i, (3 + i) % NBUF, i, True, True, True)
            return carry

        lax.fori_loop(0, (NCHUNKS - NBUF) // NBUF, main_body, 0)
        step(NCHUNKS - 2, (NCHUNKS - 2) % NBUF, 0, True, False, False)
        step(NCHUNKS - 1, (NCHUNKS - 1) % NBUF, 0, True, False, False)

        if do_den:
            pltpu.make_async_copy(w_t.at[0], den_s.at[dst_t.at[0]],
                                  sden).wait()
        for b in range(NBUF):
            pltpu.make_async_copy(rows[b], acc_s.at[dst_t.at[0]],
                                  ss[b]).wait()
        plsc.subcore_barrier()
        pltpu.sync_copy(acc_s.at[sl_own], out_ref.at[c, sl_own])

    half_pass(xplo_hbm, acclo_out, True)
    pltpu.sync_copy(den_s.at[sl_own], den_out.at[c, sl_own])
    pltpu.sync_copy(zacc_hbm, acc_s.at[sl_own])
    plsc.subcore_barrier()
    half_pass(xphi_hbm, acchi_out, False)


def _edge_stage1(srcr, dstr, asrc, adst, xplo, xphi, zacc, zden):
    mesh = plsc.VectorSubcoreMesh(core_axis_name="c", subcore_axis_name="s")
    f = pl.kernel(
        _edge1_body,
        out_type=[
            jax.ShapeDtypeStruct((NC, NPAD, CH), F32),
            jax.ShapeDtypeStruct((NC, NPAD, CH), F32),
            jax.ShapeDtypeStruct((NC, NPAD), F32),
        ],
        mesh=mesh,
        scratch_types=[
            pltpu.VMEM((NCHUNKS, CHUNK), jnp.int32),
            pltpu.VMEM((NCHUNKS, CHUNK), jnp.int32),
            pltpu.VMEM((NCHUNKS, CHUNK), F32),
            pltpu.VMEM((NPAD,), F32),
            pltpu.VMEM((NPAD,), F32),
        ] + [pltpu.VMEM((CHUNK, CH), F32)] * NBUF + [
            pltpu.VMEM_SHARED((NPAD, CH), F32),
            pltpu.VMEM_SHARED((NPAD,), F32),
        ] + [pltpu.SemaphoreType.DMA] * (2 * NBUF + 2),
        compiler_params=pltpu.CompilerParams(needs_layout_passes=False, use_tc_tiling_on_sc=False),
    )
    return f(srcr, dstr, asrc, adst, xplo, xphi, zacc, zden)


def _node2_body(alo_ref, ahi_ref, denp_ref, b1lo_ref, b1hi_ref, w2lo_ref,
                w2hi_ref, as2_ref, ad2_ref, z_ref, zs_ref, zd_ref):
    d = denp_ref[0] + denp_ref[1] + 1e-16
    hl = jnp.maximum((alo_ref[0] + alo_ref[1]) / d + b1lo_ref[...], 0.0)
    hh = jnp.maximum((ahi_ref[0] + ahi_ref[1]) / d + b1hi_ref[...], 0.0)
    z = (jnp.dot(hl, w2lo_ref[...], preferred_element_type=F32)
         + jnp.dot(hh, w2hi_ref[...], preferred_element_type=F32))
    z_ref[...] = z
    zs_ref[...] = z * as2_ref[0, 0]
    zd_ref[...] = z * ad2_ref[0, 0]


def _node_stage2(alo_p, ahi_p, den_p, b1lo, b1hi, W2lo, W2hi, as2, ad2):
    rb = 512
    grid = NPAD // rb
    return pl.pallas_call(
        _node2_body,
        grid=(grid,),
        in_specs=[
            pl.BlockSpec((NC, rb, CH), lambda i: (0, i, 0)),
            pl.BlockSpec((NC, rb, CH), lambda i: (0, i, 0)),
            pl.BlockSpec((NC, rb, 1), lambda i: (0, i, 0)),
            pl.BlockSpec((1, CH), lambda i: (0, 0)),
            pl.BlockSpec((1, CH), lambda i: (0, 0)),
            pl.BlockSpec((CH, 1), lambda i: (0, 0)),
            pl.BlockSpec((CH, 1), lambda i: (0, 0)),
            pl.BlockSpec((1, 1), lambda i: (0, 0)),
            pl.BlockSpec((1, 1), lambda i: (0, 0)),
        ],
        out_specs=[
            pl.BlockSpec((rb, 1), lambda i: (i, 0)),
            pl.BlockSpec((rb, 1), lambda i: (i, 0)),
            pl.BlockSpec((rb, 1), lambda i: (i, 0)),
        ],
        out_shape=[
            jax.ShapeDtypeStruct((NPAD, 1), F32),
            jax.ShapeDtypeStruct((NPAD, 1), F32),
            jax.ShapeDtypeStruct((NPAD, 1), F32),
        ],
    )(alo_p, ahi_p, den_p, b1lo, b1hi, W2lo, W2hi, as2, ad2)


def _edge2_body(srcr_hbm, dstr_hbm, zs_hbm, zd_hbm, z_hbm, zden_hbm,
                num_out, den_out,
                src_t, dst_t, w_t, m_t, zs_t, zd_t, z_t,
                num_s, den_s, ssc, szero):
    c = lax.axis_index("c")
    s = lax.axis_index("s")
    w_id = c * NS + s

    pltpu.async_copy(zden_hbm, num_s.at[pl.ds(s * RPT, RPT)], szero)
    pltpu.async_copy(zden_hbm, den_s.at[pl.ds(s * RPT, RPT)], szero)
    row0 = w_id * NCHUNKS
    pltpu.sync_copy(srcr_hbm.at[pl.ds(row0, NCHUNKS)], src_t)
    pltpu.sync_copy(dstr_hbm.at[pl.ds(row0, NCHUNKS)], dst_t)
    pltpu.sync_copy(zs_hbm, zs_t)
    pltpu.sync_copy(zd_hbm, zd_t)
    pltpu.sync_copy(z_hbm, z_t)

    def wk(k, carry):
        for g in range(CHUNK // 16):
            sl = pl.ds(g * 16, 16)
            isrc = src_t[k, sl]
            a = (plsc.load_gather(zs_t, [isrc])
                 + plsc.load_gather(zd_t, [dst_t[k, sl]]))
            a = jnp.where(a > 0, a, 0.2 * a)
            w = jnp.exp(a)
            w_t[k, sl] = w
            m_t[k, sl] = w * plsc.load_gather(z_t, [isrc])
        return carry

    lax.fori_loop(0, NCHUNKS, wk, 0)

    pltpu.make_async_copy(zden_hbm, num_s.at[pl.ds(s * RPT, RPT)],
                          szero).wait()
    pltpu.make_async_copy(zden_hbm, den_s.at[pl.ds(s * RPT, RPT)],
                          szero).wait()
    plsc.subcore_barrier()

    KB = 5

    def sblk(blk, carry):
        for i in range(KB):
            j = blk * KB + i
            pltpu.async_copy(m_t.at[j], num_s.at[dst_t.at[j]], ssc, add=True)
            pltpu.async_copy(w_t.at[j], den_s.at[dst_t.at[j]], ssc, add=True)
        for i in range(2 * KB):
            pltpu.make_async_copy(w_t.at[0], den_s.at[dst_t.at[0]],
                                  ssc).wait()
        return carry

    lax.fori_loop(0, NCHUNKS // KB, sblk, 0)
    plsc.subcore_barrier()
    pltpu.sync_copy(num_s.at[pl.ds(s * RPT, RPT)],
                    num_out.at[c, pl.ds(s * RPT, RPT)])
    pltpu.sync_copy(den_s.at[pl.ds(s * RPT, RPT)],
                    den_out.at[c, pl.ds(s * RPT, RPT)])


def _edge_stage2(srcr, dstr, zs, zd, z, zden):
    mesh = plsc.VectorSubcoreMesh(core_axis_name="c", subcore_axis_name="s")
    f = pl.kernel(
        _edge2_body,
        out_type=[
            jax.ShapeDtypeStruct((NC, NPAD), F32),
            jax.ShapeDtypeStruct((NC, NPAD), F32),
        ],
        mesh=mesh,
        scratch_types=[
            pltpu.VMEM((NCHUNKS, CHUNK), jnp.int32),
            pltpu.VMEM((NCHUNKS, CHUNK), jnp.int32),
            pltpu.VMEM((NCHUNKS, CHUNK), F32),
            pltpu.VMEM((NCHUNKS, CHUNK), F32),
            pltpu.VMEM((NPAD,), F32),
            pltpu.VMEM((NPAD,), F32),
            pltpu.VMEM((NPAD,), F32),
            pltpu.VMEM_SHARED((NPAD,), F32),
            pltpu.VMEM_SHARED((NPAD,), F32),
            pltpu.SemaphoreType.DMA,
            pltpu.SemaphoreType.DMA,
        ],
        compiler_params=pltpu.CompilerParams(needs_layout_passes=False, use_tc_tiling_on_sc=False),
    )
    return f(srcr, dstr, zs, zd, z, zden)


def _final_body(nump_ref, denp_ref, b2_ref, o_ref):
    o_ref[...] = ((nump_ref[0] + nump_ref[1])
                  / (denp_ref[0] + denp_ref[1] + 1e-16)) + b2_ref[0, 0]


def _final_stage(num_p, den_p, b2):
    return pl.pallas_call(
        _final_body,
        in_specs=[
            pl.BlockSpec((NC, NPAD // C, C), lambda: (0, 0, 0)),
            pl.BlockSpec((NC, NPAD // C, C), lambda: (0, 0, 0)),
            pl.BlockSpec((1, 1), lambda: (0, 0)),
        ],
        out_specs=pl.BlockSpec((NPAD // C, C), lambda: (0, 0)),
        out_shape=jax.ShapeDtypeStruct((NPAD // C, C), F32),
    )(num_p, den_p, b2)


def kernel(x, edge_index, W1, att_src1, att_dst1, b1, W2, att_src2,
           att_dst2, b2):
    x_pad = jnp.concatenate(
        [x, jnp.zeros((NPAD - N, C), F32)], axis=0)
    src = edge_index[0].astype(jnp.int32).reshape(E // CHUNK, CHUNK)
    dst = edge_index[1].astype(jnp.int32).reshape(E // CHUNK, CHUNK)
    asv = att_src1.reshape(C, 1)
    adv = att_dst1.reshape(C, 1)

    xplo, xphi, asrc, adst = _node_stage1(x_pad, W1, asv, adv)

    zacc = jnp.zeros((RPT, CH), F32)
    zden = jnp.zeros((RPT,), F32)
    alo_p, ahi_p, den_p = _edge_stage1(src, dst, asrc.reshape(NPAD),
                                       adst.reshape(NPAD), xplo, xphi,
                                       zacc, zden)

    b1f = b1.reshape(1, C)
    z, zs, zd = _node_stage2(alo_p, ahi_p, den_p.reshape(NC, NPAD, 1),
                             b1f[:, :CH], b1f[:, CH:],
                             W2[:CH], W2[CH:],
                             att_src2.reshape(1, 1), att_dst2.reshape(1, 1))

    num_p, den2_p = _edge_stage2(src, dst, zs.reshape(NPAD),
                                 zd.reshape(NPAD), z.reshape(NPAD), zden)

    o = _final_stage(num_p.reshape(NC, NPAD // C, C),
                     den2_p.reshape(NC, NPAD // C, C), b2.reshape(1, 1))
    return o.reshape(NPAD)[:N]


# speedup vs baseline: 63.1634x; 1.0143x over previous
import functools

import jax
import jax.numpy as jnp
from jax import lax
from jax.experimental import pallas as pl
from jax.experimental.pallas import tpu as pltpu
from jax.experimental.pallas import tpu_sc as plsc

F32 = jnp.float32
N = 10000
E = 320000
C = 128
NPAD = 10240
NC = 2
NS = 16
NW = NC * NS
EW = E // NW
CHUNK = 80
NCHUNKS = EW // CHUNK
RPT = NPAD // NS
CH = C // 2

_HI = lax.Precision.HIGHEST


def _node1_body(x_ref, w1_ref, asv_ref, adv_ref, xplo_ref, xphi_ref,
                as_ref, ad_ref):
    xb = jnp.dot(x_ref[...], w1_ref[...], preferred_element_type=F32)
    xplo_ref[...] = xb[:, :CH]
    xphi_ref[...] = xb[:, CH:]
    as_ref[...] = jnp.dot(xb, asv_ref[...], preferred_element_type=F32,
                          precision=_HI)
    ad_ref[...] = jnp.dot(xb, adv_ref[...], preferred_element_type=F32,
                          precision=_HI)


def _node_stage1(x_pad, W1, asv, adv):
    rb = 2048
    grid = NPAD // rb
    return pl.pallas_call(
        _node1_body,
        grid=(grid,),
        in_specs=[
            pl.BlockSpec((rb, C), lambda i: (i, 0)),
            pl.BlockSpec((C, C), lambda i: (0, 0)),
            pl.BlockSpec((C, 1), lambda i: (0, 0)),
            pl.BlockSpec((C, 1), lambda i: (0, 0)),
        ],
        out_specs=[
            pl.BlockSpec((rb, CH), lambda i: (i, 0)),
            pl.BlockSpec((rb, CH), lambda i: (i, 0)),
            pl.BlockSpec((rb, 1), lambda i: (i, 0)),
            pl.BlockSpec((rb, 1), lambda i: (i, 0)),
        ],
        out_shape=[
            jax.ShapeDtypeStruct((NPAD, CH), F32),
            jax.ShapeDtypeStruct((NPAD, CH), F32),
            jax.ShapeDtypeStruct((NPAD, 1), F32),
            jax.ShapeDtypeStruct((NPAD, 1), F32),
        ],
    )(x_pad, W1, asv, adv)


NBUF = 5


def _edge1_body(srcr_hbm, dstr_hbm, asrc_hbm, adst_hbm, xplo_hbm, xphi_hbm,
                zacc_hbm, zden_hbm, acclo_out, acchi_out, den_out,
                src_t, dst_t, w_t, asrc_t, adst_t,
                r0, r1, r2, r3, r4, acc_s, den_s,
                g0, g1, g2, g3, g4, t0, t1, t2, t3, t4, sden, szero):
    rows = [r0, r1, r2, r3, r4]
    sg = [g0, g1, g2, g3, g4]
    ss = [t0, t1, t2, t3, t4]
    c = lax.axis_index("c")
    s = lax.axis_index("s")
    w_id = c * NS + s
    sl_own = pl.ds(s * RPT, RPT)

    pltpu.async_copy(zacc_hbm, acc_s.at[sl_own], szero)
    pltpu.async_copy(zden_hbm, den_s.at[sl_own], szero)
    row0 = w_id * NCHUNKS
    pltpu.sync_copy(srcr_hbm.at[pl.ds(row0, NCHUNKS)], src_t)
    pltpu.sync_copy(dstr_hbm.at[pl.ds(row0, NCHUNKS)], dst_t)
    pltpu.sync_copy(asrc_hbm, asrc_t)
    pltpu.sync_copy(adst_hbm, adst_t)

    def wk(k, carry):
        for g in range(CHUNK // 16):
            sl = pl.ds(g * 16, 16)
            a = (plsc.load_gather(asrc_t, [src_t[k, sl]])
                 + plsc.load_gather(adst_t, [dst_t[k, sl]]))
            a = jnp.where(a > 0, a, 0.2 * a)
            w_t[k, sl] = jnp.exp(a)
        return carry

    lax.fori_loop(0, NCHUNKS, wk, 0)

    pltpu.make_async_copy(zacc_hbm, acc_s.at[sl_own], szero).wait()
    pltpu.make_async_copy(zden_hbm, den_s.at[sl_own], szero).wait()
    plsc.subcore_barrier()

    def half_pass(xp_hbm, out_ref, do_den):
        def step(j, b, b2, wait_den, wait_ss2, issue_next):
            pltpu.make_async_copy(xp_hbm.at[src_t.at[j]], rows[b],
                                  sg[b]).wait()
            jv = jnp.full((16,), j, jnp.int32)

            def scale_body(e, carry):
                wv = plsc.load_gather(w_t,
                                      [jv, jnp.full((16,), e, jnp.int32)])
                for cc in range(CH // 16):
                    sl2 = pl.ds(cc * 16, 16)
                    rows[b][e, sl2] = rows[b][e, sl2] * wv
                return carry

            lax.fori_loop(0, CHUNK, scale_body, 0, unroll=8)
            if do_den:
                if wait_den:
                    pltpu.make_async_copy(w_t.at[0], den_s.at[dst_t.at[0]],
                                          sden).wait()
                pltpu.async_copy(w_t.at[j], den_s.at[dst_t.at[j]], sden,
                                 add=True)
            pltpu.async_copy(rows[b], acc_s.at[dst_t.at[j]], ss[b], add=True)
            if issue_next:
                if wait_ss2:
                    pltpu.make_async_copy(rows[b2], acc_s.at[dst_t.at[0]],
                                          ss[b2]).wait()
                pltpu.async_copy(xp_hbm.at[src_t.at[j + 2]], rows[b2],
                                 sg[b2])

        pltpu.async_copy(xp_hbm.at[src_t.at[0]], rows[0], sg[0])
        pltpu.async_copy(xp_hbm.at[src_t.at[1]], rows[1], sg[1])
        step(0, 0, 2, False, False, True)
        step(1, 1, 3, True, False, True)
        step(2, 2, 4, True, False, True)

        def main_body(jj, carry):
            for i in range(NBUF):
                step(3 + jj * NBUF + i, (3 + i) % NBUF, i, True, True, True)
            return carry

        lax.fori_loop(0, (NCHUNKS - NBUF) // NBUF, main_body, 0)
        step(NCHUNKS - 2, (NCHUNKS - 2) % NBUF, 0, True, False, False)
        step(NCHUNKS - 1, (NCHUNKS - 1) % NBUF, 0, True, False, False)

        if do_den:
            pltpu.make_async_copy(w_t.at[0], den_s.at[dst_t.at[0]],
                                  sden).wait()
        for b in range(NBUF):
            pltpu.make_async_copy(rows[b], acc_s.at[dst_t.at[0]],
                                  ss[b]).wait()
        plsc.subcore_barrier()
        pltpu.sync_copy(acc_s.at[sl_own], out_ref.at[c, sl_own])

    half_pass(xplo_hbm, acclo_out, True)
    pltpu.sync_copy(den_s.at[sl_own], den_out.at[c, sl_own])
    pltpu.sync_copy(zacc_hbm, acc_s.at[sl_own])
    plsc.subcore_barrier()
    half_pass(xphi_hbm, acchi_out, False)


def _edge_stage1(srcr, dstr, asrc, adst, xplo, xphi, zacc, zden):
    mesh = plsc.VectorSubcoreMesh(core_axis_name="c", subcore_axis_name="s")
    f = pl.kernel(
        _edge1_body,
        out_type=[
            jax.ShapeDtypeStruct((NC, NPAD, CH), F32),
            jax.ShapeDtypeStruct((NC, NPAD, CH), F32),
            jax.ShapeDtypeStruct((NC, NPAD), F32),
        ],
        mesh=mesh,
        scratch_types=[
            pltpu.VMEM((NCHUNKS, CHUNK), jnp.int32),
            pltpu.VMEM((NCHUNKS, CHUNK), jnp.int32),
            pltpu.VMEM((NCHUNKS, CHUNK), F32),
            pltpu.VMEM((NPAD,), F32),
            pltpu.VMEM((NPAD,), F32),
        ] + [pltpu.VMEM((CHUNK, CH), F32)] * NBUF + [
            pltpu.VMEM_SHARED((NPAD, CH), F32),
            pltpu.VMEM_SHARED((NPAD,), F32),
        ] + [pltpu.SemaphoreType.DMA] * (2 * NBUF + 2),
        compiler_params=pltpu.CompilerParams(needs_layout_passes=False, use_tc_tiling_on_sc=False),
    )
    return f(srcr, dstr, asrc, adst, xplo, xphi, zacc, zden)


def _node2_body(alo_ref, ahi_ref, denp_ref, b1lo_ref, b1hi_ref, w2lo_ref,
                w2hi_ref, as2_ref, ad2_ref, z_ref, zs_ref, zd_ref):
    d = denp_ref[0] + denp_ref[1] + 1e-16
    hl = jnp.maximum((alo_ref[0] + alo_ref[1]) / d + b1lo_ref[...], 0.0)
    hh = jnp.maximum((ahi_ref[0] + ahi_ref[1]) / d + b1hi_ref[...], 0.0)
    z = (jnp.dot(hl, w2lo_ref[...], preferred_element_type=F32)
         + jnp.dot(hh, w2hi_ref[...], preferred_element_type=F32))
    z_ref[...] = z
    zs_ref[...] = z * as2_ref[0, 0]
    zd_ref[...] = z * ad2_ref[0, 0]


def _node_stage2(alo_p, ahi_p, den_p, b1lo, b1hi, W2lo, W2hi, as2, ad2):
    rb = 2048
    grid = NPAD // rb
    return pl.pallas_call(
        _node2_body,
        grid=(grid,),
        in_specs=[
            pl.BlockSpec((NC, rb, CH), lambda i: (0, i, 0)),
            pl.BlockSpec((NC, rb, CH), lambda i: (0, i, 0)),
            pl.BlockSpec((NC, rb, 1), lambda i: (0, i, 0)),
            pl.BlockSpec((1, CH), lambda i: (0, 0)),
            pl.BlockSpec((1, CH), lambda i: (0, 0)),
            pl.BlockSpec((CH, 1), lambda i: (0, 0)),
            pl.BlockSpec((CH, 1), lambda i: (0, 0)),
            pl.BlockSpec((1, 1), lambda i: (0, 0)),
            pl.BlockSpec((1, 1), lambda i: (0, 0)),
        ],
        out_specs=[
            pl.BlockSpec((rb, 1), lambda i: (i, 0)),
            pl.BlockSpec((rb, 1), lambda i: (i, 0)),
            pl.BlockSpec((rb, 1), lambda i: (i, 0)),
        ],
        out_shape=[
            jax.ShapeDtypeStruct((NPAD, 1), F32),
            jax.ShapeDtypeStruct((NPAD, 1), F32),
            jax.ShapeDtypeStruct((NPAD, 1), F32),
        ],
    )(alo_p, ahi_p, den_p, b1lo, b1hi, W2lo, W2hi, as2, ad2)


def _edge2_body(srcr_hbm, dstr_hbm, zs_hbm, zd_hbm, z_hbm, zden_hbm,
                num_out, den_out,
                src_t, dst_t, w_t, m_t, zs_t, zd_t, z_t,
                num_s, den_s, ssc, szero):
    c = lax.axis_index("c")
    s = lax.axis_index("s")
    w_id = c * NS + s

    pltpu.async_copy(zden_hbm, num_s.at[pl.ds(s * RPT, RPT)], szero)
    pltpu.async_copy(zden_hbm, den_s.at[pl.ds(s * RPT, RPT)], szero)
    row0 = w_id * NCHUNKS
    pltpu.sync_copy(srcr_hbm.at[pl.ds(row0, NCHUNKS)], src_t)
    pltpu.sync_copy(dstr_hbm.at[pl.ds(row0, NCHUNKS)], dst_t)
    pltpu.sync_copy(zs_hbm, zs_t)
    pltpu.sync_copy(zd_hbm, zd_t)
    pltpu.sync_copy(z_hbm, z_t)

    def wk(k, carry):
        for g in range(CHUNK // 16):
            sl = pl.ds(g * 16, 16)
            isrc = src_t[k, sl]
            a = (plsc.load_gather(zs_t, [isrc])
                 + plsc.load_gather(zd_t, [dst_t[k, sl]]))
            a = jnp.where(a > 0, a, 0.2 * a)
            w = jnp.exp(a)
            w_t[k, sl] = w
            m_t[k, sl] = w * plsc.load_gather(z_t, [isrc])
        return carry

    lax.fori_loop(0, NCHUNKS, wk, 0)

    pltpu.make_async_copy(zden_hbm, num_s.at[pl.ds(s * RPT, RPT)],
                          szero).wait()
    pltpu.make_async_copy(zden_hbm, den_s.at[pl.ds(s * RPT, RPT)],
                          szero).wait()
    plsc.subcore_barrier()

    KB = 5

    def sblk(blk, carry):
        for i in range(KB):
            j = blk * KB + i
            pltpu.async_copy(m_t.at[j], num_s.at[dst_t.at[j]], ssc, add=True)
            pltpu.async_copy(w_t.at[j], den_s.at[dst_t.at[j]], ssc, add=True)
        for i in range(2 * KB):
            pltpu.make_async_copy(w_t.at[0], den_s.at[dst_t.at[0]],
                                  ssc).wait()
        return carry

    lax.fori_loop(0, NCHUNKS // KB, sblk, 0)
    plsc.subcore_barrier()
    pltpu.sync_copy(num_s.at[pl.ds(s * RPT, RPT)],
                    num_out.at[c, pl.ds(s * RPT, RPT)])
    pltpu.sync_copy(den_s.at[pl.ds(s * RPT, RPT)],
                    den_out.at[c, pl.ds(s * RPT, RPT)])


def _edge_stage2(srcr, dstr, zs, zd, z, zden):
    mesh = plsc.VectorSubcoreMesh(core_axis_name="c", subcore_axis_name="s")
    f = pl.kernel(
        _edge2_body,
        out_type=[
            jax.ShapeDtypeStruct((NC, NPAD), F32),
            jax.ShapeDtypeStruct((NC, NPAD), F32),
        ],
        mesh=mesh,
        scratch_types=[
            pltpu.VMEM((NCHUNKS, CHUNK), jnp.int32),
            pltpu.VMEM((NCHUNKS, CHUNK), jnp.int32),
            pltpu.VMEM((NCHUNKS, CHUNK), F32),
            pltpu.VMEM((NCHUNKS, CHUNK), F32),
            pltpu.VMEM((NPAD,), F32),
            pltpu.VMEM((NPAD,), F32),
            pltpu.VMEM((NPAD,), F32),
            pltpu.VMEM_SHARED((NPAD,), F32),
            pltpu.VMEM_SHARED((NPAD,), F32),
            pltpu.SemaphoreType.DMA,
            pltpu.SemaphoreType.DMA,
        ],
        compiler_params=pltpu.CompilerParams(needs_layout_passes=False, use_tc_tiling_on_sc=False),
    )
    return f(srcr, dstr, zs, zd, z, zden)


def _final_body(nump_ref, denp_ref, b2_ref, o_ref):
    o_ref[...] = ((nump_ref[0] + nump_ref[1])
                  / (denp_ref[0] + denp_ref[1] + 1e-16)) + b2_ref[0, 0]


def _final_stage(num_p, den_p, b2):
    return pl.pallas_call(
        _final_body,
        in_specs=[
            pl.BlockSpec((NC, NPAD // C, C), lambda: (0, 0, 0)),
            pl.BlockSpec((NC, NPAD // C, C), lambda: (0, 0, 0)),
            pl.BlockSpec((1, 1), lambda: (0, 0)),
        ],
        out_specs=pl.BlockSpec((NPAD // C, C), lambda: (0, 0)),
        out_shape=jax.ShapeDtypeStruct((NPAD // C, C), F32),
    )(num_p, den_p, b2)


def kernel(x, edge_index, W1, att_src1, att_dst1, b1, W2, att_src2,
           att_dst2, b2):
    x_pad = jnp.concatenate(
        [x, jnp.zeros((NPAD - N, C), F32)], axis=0)
    src = edge_index[0].astype(jnp.int32).reshape(E // CHUNK, CHUNK)
    dst = edge_index[1].astype(jnp.int32).reshape(E // CHUNK, CHUNK)
    asv = att_src1.reshape(C, 1)
    adv = att_dst1.reshape(C, 1)

    xplo, xphi, asrc, adst = _node_stage1(x_pad, W1, asv, adv)

    zacc = jnp.zeros((RPT, CH), F32)
    zden = jnp.zeros((RPT,), F32)
    alo_p, ahi_p, den_p = _edge_stage1(src, dst, asrc.reshape(NPAD),
                                       adst.reshape(NPAD), xplo, xphi,
                                       zacc, zden)

    b1f = b1.reshape(1, C)
    z, zs, zd = _node_stage2(alo_p, ahi_p, den_p.reshape(NC, NPAD, 1),
                             b1f[:, :CH], b1f[:, CH:],
                             W2[:CH], W2[CH:],
                             att_src2.reshape(1, 1), att_dst2.reshape(1, 1))

    num_p, den2_p = _edge_stage2(src, dst, zs.reshape(NPAD),
                                 zd.reshape(NPAD), z.reshape(NPAD), zden)

    o = _final_stage(num_p.reshape(NC, NPAD // C, C),
                     den2_p.reshape(NC, NPAD // C, C), b2.reshape(1, 1))
    return o.reshape(NPAD)[:N]


# speedup vs baseline: 68.4960x; 1.0844x over previous
import functools

import jax
import jax.numpy as jnp
from jax import lax
from jax.experimental import pallas as pl
from jax.experimental.pallas import tpu as pltpu
from jax.experimental.pallas import tpu_sc as plsc

F32 = jnp.float32
N = 10000
E = 320000
C = 128
NPAD = 10240
NC = 2
NS = 16
NW = NC * NS
EW = E // NW
CHUNK = 80
NCHUNKS = EW // CHUNK
RPT = NPAD // NS
CH = C // 2

_HI = lax.Precision.HIGHEST


def _node1_body(x_ref, w1_ref, asv_ref, adv_ref, xplo_ref, xphi_ref,
                as_ref, ad_ref):
    xb = jnp.dot(x_ref[...], w1_ref[...], preferred_element_type=F32)
    xplo_ref[...] = xb[:, :CH]
    xphi_ref[...] = xb[:, CH:]
    as_ref[...] = jnp.dot(xb, asv_ref[...], preferred_element_type=F32,
                          precision=_HI)
    ad_ref[...] = jnp.dot(xb, adv_ref[...], preferred_element_type=F32,
                          precision=_HI)


def _node_stage1(x_pad, W1, asv, adv):
    rb = 2048
    grid = NPAD // rb
    return pl.pallas_call(
        _node1_body,
        grid=(grid,),
        in_specs=[
            pl.BlockSpec((rb, C), lambda i: (i, 0)),
            pl.BlockSpec((C, C), lambda i: (0, 0)),
            pl.BlockSpec((C, 1), lambda i: (0, 0)),
            pl.BlockSpec((C, 1), lambda i: (0, 0)),
        ],
        out_specs=[
            pl.BlockSpec((rb, CH), lambda i: (i, 0)),
            pl.BlockSpec((rb, CH), lambda i: (i, 0)),
            pl.BlockSpec((rb, 1), lambda i: (i, 0)),
            pl.BlockSpec((rb, 1), lambda i: (i, 0)),
        ],
        out_shape=[
            jax.ShapeDtypeStruct((NPAD, CH), F32),
            jax.ShapeDtypeStruct((NPAD, CH), F32),
            jax.ShapeDtypeStruct((NPAD, 1), F32),
            jax.ShapeDtypeStruct((NPAD, 1), F32),
        ],
    )(x_pad, W1, asv, adv)


NBUF = 5


def _edge1_body(srcr_hbm, dstr_hbm, asrc_hbm, adst_hbm, xplo_hbm, xphi_hbm,
                zacc_hbm, zden_hbm, acclo_out, acchi_out, den_out,
                src_t, dst_t, w_t, asrc_t, adst_t,
                r0, r1, r2, r3, r4, acc_s, den_s,
                g0, g1, g2, g3, g4, t0, t1, t2, t3, t4, sden, szero):
    rows = [r0, r1, r2, r3, r4]
    sg = [g0, g1, g2, g3, g4]
    ss = [t0, t1, t2, t3, t4]
    c = lax.axis_index("c")
    s = lax.axis_index("s")
    w_id = c * NS + s
    sl_own = pl.ds(s * RPT, RPT)

    pltpu.async_copy(zacc_hbm, acc_s.at[sl_own], szero)
    pltpu.async_copy(zden_hbm, den_s.at[sl_own], szero)
    row0 = w_id * NCHUNKS
    pltpu.sync_copy(srcr_hbm.at[pl.ds(row0, NCHUNKS)], src_t)
    pltpu.sync_copy(dstr_hbm.at[pl.ds(row0, NCHUNKS)], dst_t)
    pltpu.sync_copy(asrc_hbm, asrc_t)
    pltpu.sync_copy(adst_hbm, adst_t)

    def wk(k, carry):
        for g in range(CHUNK // 16):
            sl = pl.ds(g * 16, 16)
            a = (plsc.load_gather(asrc_t, [src_t[k, sl]])
                 + plsc.load_gather(adst_t, [dst_t[k, sl]]))
            a = jnp.where(a > 0, a, 0.2 * a)
            w_t[k, sl] = jnp.exp(a)
        return carry

    lax.fori_loop(0, NCHUNKS, wk, 0)

    pltpu.make_async_copy(zacc_hbm, acc_s.at[sl_own], szero).wait()
    pltpu.make_async_copy(zden_hbm, den_s.at[sl_own], szero).wait()
    plsc.subcore_barrier()

    def half_pass(xp_hbm, out_ref, do_den):
        def step(j, b, b2, wait_den, wait_ss2, issue_next):
            if issue_next:
                if wait_ss2:
                    pltpu.make_async_copy(rows[b2], acc_s.at[dst_t.at[0]],
                                          ss[b2]).wait()
                pltpu.async_copy(xp_hbm.at[src_t.at[j + 2]], rows[b2],
                                 sg[b2])
            pltpu.make_async_copy(xp_hbm.at[src_t.at[j]], rows[b],
                                  sg[b]).wait()
            jv = jnp.full((16,), j, jnp.int32)

            def scale_body(e, carry):
                wv = plsc.load_gather(w_t,
                                      [jv, jnp.full((16,), e, jnp.int32)])
                for cc in range(CH // 16):
                    sl2 = pl.ds(cc * 16, 16)
                    rows[b][e, sl2] = rows[b][e, sl2] * wv
                return carry

            lax.fori_loop(0, CHUNK, scale_body, 0, unroll=8)
            if do_den:
                if wait_den:
                    pltpu.make_async_copy(w_t.at[0], den_s.at[dst_t.at[0]],
                                          sden).wait()
                pltpu.async_copy(w_t.at[j], den_s.at[dst_t.at[j]], sden,
                                 add=True)
            pltpu.async_copy(rows[b], acc_s.at[dst_t.at[j]], ss[b], add=True)

        pltpu.async_copy(xp_hbm.at[src_t.at[0]], rows[0], sg[0])
        pltpu.async_copy(xp_hbm.at[src_t.at[1]], rows[1], sg[1])
        step(0, 0, 2, False, False, True)
        step(1, 1, 3, True, False, True)
        step(2, 2, 4, True, False, True)

        def main_body(jj, carry):
            for i in range(NBUF):
                step(3 + jj * NBUF + i, (3 + i) % NBUF, i, True, True, True)
            return carry

        lax.fori_loop(0, (NCHUNKS - NBUF) // NBUF, main_body, 0)
        step(NCHUNKS - 2, (NCHUNKS - 2) % NBUF, 0, True, False, False)
        step(NCHUNKS - 1, (NCHUNKS - 1) % NBUF, 0, True, False, False)

        if do_den:
            pltpu.make_async_copy(w_t.at[0], den_s.at[dst_t.at[0]],
                                  sden).wait()
        for b in range(NBUF):
            pltpu.make_async_copy(rows[b], acc_s.at[dst_t.at[0]],
                                  ss[b]).wait()
        plsc.subcore_barrier()
        pltpu.sync_copy(acc_s.at[sl_own], out_ref.at[c, sl_own])

    half_pass(xplo_hbm, acclo_out, True)
    pltpu.sync_copy(den_s.at[sl_own], den_out.at[c, sl_own])
    pltpu.sync_copy(zacc_hbm, acc_s.at[sl_own])
    plsc.subcore_barrier()
    half_pass(xphi_hbm, acchi_out, False)


def _edge_stage1(srcr, dstr, asrc, adst, xplo, xphi, zacc, zden):
    mesh = plsc.VectorSubcoreMesh(core_axis_name="c", subcore_axis_name="s")
    f = pl.kernel(
        _edge1_body,
        out_type=[
            jax.ShapeDtypeStruct((NC, NPAD, CH), F32),
            jax.ShapeDtypeStruct((NC, NPAD, CH), F32),
            jax.ShapeDtypeStruct((NC, NPAD), F32),
        ],
        mesh=mesh,
        scratch_types=[
            pltpu.VMEM((NCHUNKS, CHUNK), jnp.int32),
            pltpu.VMEM((NCHUNKS, CHUNK), jnp.int32),
            pltpu.VMEM((NCHUNKS, CHUNK), F32),
            pltpu.VMEM((NPAD,), F32),
            pltpu.VMEM((NPAD,), F32),
        ] + [pltpu.VMEM((CHUNK, CH), F32)] * NBUF + [
            pltpu.VMEM_SHARED((NPAD, CH), F32),
            pltpu.VMEM_SHARED((NPAD,), F32),
        ] + [pltpu.SemaphoreType.DMA] * (2 * NBUF + 2),
        compiler_params=pltpu.CompilerParams(needs_layout_passes=False, use_tc_tiling_on_sc=False),
    )
    return f(srcr, dstr, asrc, adst, xplo, xphi, zacc, zden)


def _node2_body(alo_ref, ahi_ref, denp_ref, b1lo_ref, b1hi_ref, w2lo_ref,
                w2hi_ref, as2_ref, ad2_ref, z_ref, zs_ref, zd_ref):
    d = denp_ref[0] + denp_ref[1] + 1e-16
    hl = jnp.maximum((alo_ref[0] + alo_ref[1]) / d + b1lo_ref[...], 0.0)
    hh = jnp.maximum((ahi_ref[0] + ahi_ref[1]) / d + b1hi_ref[...], 0.0)
    z = (jnp.dot(hl, w2lo_ref[...], preferred_element_type=F32)
         + jnp.dot(hh, w2hi_ref[...], preferred_element_type=F32))
    z_ref[...] = z
    zs_ref[...] = z * as2_ref[0, 0]
    zd_ref[...] = z * ad2_ref[0, 0]


def _node_stage2(alo_p, ahi_p, den_p, b1lo, b1hi, W2lo, W2hi, as2, ad2):
    rb = 2048
    grid = NPAD // rb
    return pl.pallas_call(
        _node2_body,
        grid=(grid,),
        in_specs=[
            pl.BlockSpec((NC, rb, CH), lambda i: (0, i, 0)),
            pl.BlockSpec((NC, rb, CH), lambda i: (0, i, 0)),
            pl.BlockSpec((NC, rb, 1), lambda i: (0, i, 0)),
            pl.BlockSpec((1, CH), lambda i: (0, 0)),
            pl.BlockSpec((1, CH), lambda i: (0, 0)),
            pl.BlockSpec((CH, 1), lambda i: (0, 0)),
            pl.BlockSpec((CH, 1), lambda i: (0, 0)),
            pl.BlockSpec((1, 1), lambda i: (0, 0)),
            pl.BlockSpec((1, 1), lambda i: (0, 0)),
        ],
        out_specs=[
            pl.BlockSpec((rb, 1), lambda i: (i, 0)),
            pl.BlockSpec((rb, 1), lambda i: (i, 0)),
            pl.BlockSpec((rb, 1), lambda i: (i, 0)),
        ],
        out_shape=[
            jax.ShapeDtypeStruct((NPAD, 1), F32),
            jax.ShapeDtypeStruct((NPAD, 1), F32),
            jax.ShapeDtypeStruct((NPAD, 1), F32),
        ],
    )(alo_p, ahi_p, den_p, b1lo, b1hi, W2lo, W2hi, as2, ad2)


def _edge2_body(srcr_hbm, dstr_hbm, zs_hbm, zd_hbm, z_hbm, zden_hbm,
                num_out, den_out,
                src_t, dst_t, w_t, m_t, zs_t, zd_t, z_t,
                num_s, den_s, ssc, szero):
    c = lax.axis_index("c")
    s = lax.axis_index("s")
    w_id = c * NS + s

    pltpu.async_copy(zden_hbm, num_s.at[pl.ds(s * RPT, RPT)], szero)
    pltpu.async_copy(zden_hbm, den_s.at[pl.ds(s * RPT, RPT)], szero)
    row0 = w_id * NCHUNKS
    pltpu.sync_copy(srcr_hbm.at[pl.ds(row0, NCHUNKS)], src_t)
    pltpu.sync_copy(dstr_hbm.at[pl.ds(row0, NCHUNKS)], dst_t)
    pltpu.sync_copy(zs_hbm, zs_t)
    pltpu.sync_copy(zd_hbm, zd_t)
    pltpu.sync_copy(z_hbm, z_t)

    def wk(k, carry):
        for g in range(CHUNK // 16):
            sl = pl.ds(g * 16, 16)
            isrc = src_t[k, sl]
            a = (plsc.load_gather(zs_t, [isrc])
                 + plsc.load_gather(zd_t, [dst_t[k, sl]]))
            a = jnp.where(a > 0, a, 0.2 * a)
            w = jnp.exp(a)
            w_t[k, sl] = w
            m_t[k, sl] = w * plsc.load_gather(z_t, [isrc])
        return carry

    lax.fori_loop(0, NCHUNKS, wk, 0)

    pltpu.make_async_copy(zden_hbm, num_s.at[pl.ds(s * RPT, RPT)],
                          szero).wait()
    pltpu.make_async_copy(zden_hbm, den_s.at[pl.ds(s * RPT, RPT)],
                          szero).wait()
    plsc.subcore_barrier()

    KB = 5

    def sblk(blk, carry):
        for i in range(KB):
            j = blk * KB + i
            pltpu.async_copy(m_t.at[j], num_s.at[dst_t.at[j]], ssc, add=True)
            pltpu.async_copy(w_t.at[j], den_s.at[dst_t.at[j]], ssc, add=True)
        for i in range(2 * KB):
            pltpu.make_async_copy(w_t.at[0], den_s.at[dst_t.at[0]],
                                  ssc).wait()
        return carry

    lax.fori_loop(0, NCHUNKS // KB, sblk, 0)
    plsc.subcore_barrier()
    pltpu.sync_copy(num_s.at[pl.ds(s * RPT, RPT)],
                    num_out.at[c, pl.ds(s * RPT, RPT)])
    pltpu.sync_copy(den_s.at[pl.ds(s * RPT, RPT)],
                    den_out.at[c, pl.ds(s * RPT, RPT)])


def _edge_stage2(srcr, dstr, zs, zd, z, zden):
    mesh = plsc.VectorSubcoreMesh(core_axis_name="c", subcore_axis_name="s")
    f = pl.kernel(
        _edge2_body,
        out_type=[
            jax.ShapeDtypeStruct((NC, NPAD), F32),
            jax.ShapeDtypeStruct((NC, NPAD), F32),
        ],
        mesh=mesh,
        scratch_types=[
            pltpu.VMEM((NCHUNKS, CHUNK), jnp.int32),
            pltpu.VMEM((NCHUNKS, CHUNK), jnp.int32),
            pltpu.VMEM((NCHUNKS, CHUNK), F32),
            pltpu.VMEM((NCHUNKS, CHUNK), F32),
            pltpu.VMEM((NPAD,), F32),
            pltpu.VMEM((NPAD,), F32),
            pltpu.VMEM((NPAD,), F32),
            pltpu.VMEM_SHARED((NPAD,), F32),
            pltpu.VMEM_SHARED((NPAD,), F32),
            pltpu.SemaphoreType.DMA,
            pltpu.SemaphoreType.DMA,
        ],
        compiler_params=pltpu.CompilerParams(needs_layout_passes=False, use_tc_tiling_on_sc=False),
    )
    return f(srcr, dstr, zs, zd, z, zden)


def _final_body(nump_ref, denp_ref, b2_ref, o_ref):
    o_ref[...] = ((nump_ref[0] + nump_ref[1])
                  / (denp_ref[0] + denp_ref[1] + 1e-16)) + b2_ref[0, 0]


def _final_stage(num_p, den_p, b2):
    return pl.pallas_call(
        _final_body,
        in_specs=[
            pl.BlockSpec((NC, NPAD // C, C), lambda: (0, 0, 0)),
            pl.BlockSpec((NC, NPAD // C, C), lambda: (0, 0, 0)),
            pl.BlockSpec((1, 1), lambda: (0, 0)),
        ],
        out_specs=pl.BlockSpec((NPAD // C, C), lambda: (0, 0)),
        out_shape=jax.ShapeDtypeStruct((NPAD // C, C), F32),
    )(num_p, den_p, b2)


def kernel(x, edge_index, W1, att_src1, att_dst1, b1, W2, att_src2,
           att_dst2, b2):
    x_pad = jnp.concatenate(
        [x, jnp.zeros((NPAD - N, C), F32)], axis=0)
    src = edge_index[0].astype(jnp.int32).reshape(E // CHUNK, CHUNK)
    dst = edge_index[1].astype(jnp.int32).reshape(E // CHUNK, CHUNK)
    asv = att_src1.reshape(C, 1)
    adv = att_dst1.reshape(C, 1)

    xplo, xphi, asrc, adst = _node_stage1(x_pad, W1, asv, adv)

    zacc = jnp.zeros((RPT, CH), F32)
    zden = jnp.zeros((RPT,), F32)
    alo_p, ahi_p, den_p = _edge_stage1(src, dst, asrc.reshape(NPAD),
                                       adst.reshape(NPAD), xplo, xphi,
                                       zacc, zden)

    b1f = b1.reshape(1, C)
    z, zs, zd = _node_stage2(alo_p, ahi_p, den_p.reshape(NC, NPAD, 1),
                             b1f[:, :CH], b1f[:, CH:],
                             W2[:CH], W2[CH:],
                             att_src2.reshape(1, 1), att_dst2.reshape(1, 1))

    num_p, den2_p = _edge_stage2(src, dst, zs.reshape(NPAD),
                                 zd.reshape(NPAD), z.reshape(NPAD), zden)

    o = _final_stage(num_p.reshape(NC, NPAD // C, C),
                     den2_p.reshape(NC, NPAD // C, C), b2.reshape(1, 1))
    return o.reshape(NPAD)[:N]


# speedup vs baseline: 71.3736x; 1.0420x over previous
import functools

import jax
import jax.numpy as jnp
from jax import lax
from jax.experimental import pallas as pl
from jax.experimental.pallas import tpu as pltpu
from jax.experimental.pallas import tpu_sc as plsc

F32 = jnp.float32
N = 10000
E = 320000
C = 128
NPAD = 10240
NC = 2
NS = 16
NW = NC * NS
EW = E // NW
CHUNK = 80
NCHUNKS = EW // CHUNK
RPT = NPAD // NS
CH = C // 2

_HI = lax.Precision.HIGHEST


def _node1_body(x_ref, w1_ref, asv_ref, adv_ref, xplo_ref, xphi_ref,
                as_ref, ad_ref):
    xb = jnp.dot(x_ref[...], w1_ref[...], preferred_element_type=F32)
    xplo_ref[...] = xb[:, :CH]
    xphi_ref[...] = xb[:, CH:]
    as_ref[...] = jnp.dot(xb, asv_ref[...], preferred_element_type=F32,
                          precision=_HI)
    ad_ref[...] = jnp.dot(xb, adv_ref[...], preferred_element_type=F32,
                          precision=_HI)


def _node_stage1(x_pad, W1, asv, adv):
    rb = 2048
    grid = NPAD // rb
    return pl.pallas_call(
        _node1_body,
        grid=(grid,),
        in_specs=[
            pl.BlockSpec((rb, C), lambda i: (i, 0)),
            pl.BlockSpec((C, C), lambda i: (0, 0)),
            pl.BlockSpec((C, 1), lambda i: (0, 0)),
            pl.BlockSpec((C, 1), lambda i: (0, 0)),
        ],
        out_specs=[
            pl.BlockSpec((rb, CH), lambda i: (i, 0)),
            pl.BlockSpec((rb, CH), lambda i: (i, 0)),
            pl.BlockSpec((rb, 1), lambda i: (i, 0)),
            pl.BlockSpec((rb, 1), lambda i: (i, 0)),
        ],
        out_shape=[
            jax.ShapeDtypeStruct((NPAD, CH), F32),
            jax.ShapeDtypeStruct((NPAD, CH), F32),
            jax.ShapeDtypeStruct((NPAD, 1), F32),
            jax.ShapeDtypeStruct((NPAD, 1), F32),
        ],
    )(x_pad, W1, asv, adv)


NBUF = 5


def _edge1_body(srcr_hbm, dstr_hbm, asrc_hbm, adst_hbm, xplo_hbm, xphi_hbm,
                zacc_hbm, zden_hbm, acclo_out, acchi_out, den_out,
                src_t, dst_t, w_t, asrc_t, adst_t,
                r0, r1, r2, r3, r4, acc_s, den_s,
                g0, g1, g2, g3, g4, t0, t1, t2, t3, t4, sden, szero):
    rows = [r0, r1, r2, r3, r4]
    sg = [g0, g1, g2, g3, g4]
    ss = [t0, t1, t2, t3, t4]
    c = lax.axis_index("c")
    s = lax.axis_index("s")
    w_id = c * NS + s
    sl_own = pl.ds(s * RPT, RPT)

    pltpu.async_copy(zacc_hbm, acc_s.at[sl_own], szero)
    pltpu.async_copy(zden_hbm, den_s.at[sl_own], szero)
    row0 = w_id * NCHUNKS
    pltpu.sync_copy(srcr_hbm.at[pl.ds(row0, NCHUNKS)], src_t)
    pltpu.sync_copy(dstr_hbm.at[pl.ds(row0, NCHUNKS)], dst_t)
    pltpu.sync_copy(asrc_hbm, asrc_t)
    pltpu.sync_copy(adst_hbm, adst_t)

    def wk(k, carry):
        for g in range(CHUNK // 16):
            sl = pl.ds(g * 16, 16)
            a = (plsc.load_gather(asrc_t, [src_t[k, sl]])
                 + plsc.load_gather(adst_t, [dst_t[k, sl]]))
            a = jnp.where(a > 0, a, 0.2 * a)
            w_t[k, sl] = jnp.exp(a)
        return carry

    lax.fori_loop(0, NCHUNKS, wk, 0)

    pltpu.make_async_copy(zacc_hbm, acc_s.at[sl_own], szero).wait()
    pltpu.make_async_copy(zden_hbm, den_s.at[sl_own], szero).wait()
    plsc.subcore_barrier()

    def half_pass(xp_hbm, out_ref, do_den):
        def step(j, b, b2, wait_den, wait_ss2, issue_next):
            if issue_next:
                if wait_ss2:
                    pltpu.make_async_copy(rows[b2], acc_s.at[dst_t.at[0]],
                                          ss[b2]).wait()
                pltpu.async_copy(xp_hbm.at[src_t.at[j + 2]], rows[b2],
                                 sg[b2])
            pltpu.make_async_copy(xp_hbm.at[src_t.at[j]], rows[b],
                                  sg[b]).wait()
            jv = jnp.full((16,), j, jnp.int32)

            def scale_body(e, carry):
                wv = plsc.load_gather(w_t,
                                      [jv, jnp.full((16,), e, jnp.int32)])
                for cc in range(CH // 16):
                    sl2 = pl.ds(cc * 16, 16)
                    rows[b][e, sl2] = rows[b][e, sl2] * wv
                return carry

            lax.fori_loop(0, CHUNK, scale_body, 0, unroll=8)
            if do_den:
                if wait_den:
                    pltpu.make_async_copy(w_t.at[0], den_s.at[dst_t.at[0]],
                                          sden).wait()
                pltpu.async_copy(w_t.at[j], den_s.at[dst_t.at[j]], sden,
                                 add=True)
            pltpu.async_copy(rows[b], acc_s.at[dst_t.at[j]], ss[b], add=True)

        pltpu.async_copy(xp_hbm.at[src_t.at[0]], rows[0], sg[0])
        pltpu.async_copy(xp_hbm.at[src_t.at[1]], rows[1], sg[1])
        step(0, 0, 2, False, False, True)
        step(1, 1, 3, True, False, True)
        step(2, 2, 4, True, False, True)

        def main_body(jj, carry):
            for i in range(NBUF):
                step(3 + jj * NBUF + i, (3 + i) % NBUF, i, True, True, True)
            return carry

        lax.fori_loop(0, (NCHUNKS - NBUF) // NBUF, main_body, 0)
        step(NCHUNKS - 2, (NCHUNKS - 2) % NBUF, 0, True, False, False)
        step(NCHUNKS - 1, (NCHUNKS - 1) % NBUF, 0, True, False, False)

        if do_den:
            pltpu.make_async_copy(w_t.at[0], den_s.at[dst_t.at[0]],
                                  sden).wait()
        for b in range(NBUF):
            pltpu.make_async_copy(rows[b], acc_s.at[dst_t.at[0]],
                                  ss[b]).wait()
        plsc.subcore_barrier()
        pltpu.sync_copy(acc_s.at[sl_own], out_ref.at[c, sl_own])

    half_pass(xplo_hbm, acclo_out, True)
    pltpu.sync_copy(den_s.at[sl_own], den_out.at[c, sl_own])
    pltpu.sync_copy(zacc_hbm, acc_s.at[sl_own])
    plsc.subcore_barrier()
    half_pass(xphi_hbm, acchi_out, False)


def _edge_stage1(srcr, dstr, asrc, adst, xplo, xphi, zacc, zden):
    mesh = plsc.VectorSubcoreMesh(core_axis_name="c", subcore_axis_name="s")
    f = pl.kernel(
        _edge1_body,
        out_type=[
            jax.ShapeDtypeStruct((NC, NPAD, CH), F32),
            jax.ShapeDtypeStruct((NC, NPAD, CH), F32),
            jax.ShapeDtypeStruct((NC, NPAD), F32),
        ],
        mesh=mesh,
        scratch_types=[
            pltpu.VMEM((NCHUNKS, CHUNK), jnp.int32),
            pltpu.VMEM((NCHUNKS, CHUNK), jnp.int32),
            pltpu.VMEM((NCHUNKS, CHUNK), F32),
            pltpu.VMEM((NPAD,), F32),
            pltpu.VMEM((NPAD,), F32),
        ] + [pltpu.VMEM((CHUNK, CH), F32)] * NBUF + [
            pltpu.VMEM_SHARED((NPAD, CH), F32),
            pltpu.VMEM_SHARED((NPAD,), F32),
        ] + [pltpu.SemaphoreType.DMA] * (2 * NBUF + 2),
        compiler_params=pltpu.CompilerParams(needs_layout_passes=False, use_tc_tiling_on_sc=False),
    )
    return f(srcr, dstr, asrc, adst, xplo, xphi, zacc, zden)


def _node2_body(alo_ref, ahi_ref, denp_ref, b1lo_ref, b1hi_ref, w2lo_ref,
                w2hi_ref, z_ref):
    d = denp_ref[0] + denp_ref[1] + 1e-16
    hl = jnp.maximum((alo_ref[0] + alo_ref[1]) / d + b1lo_ref[...], 0.0)
    hh = jnp.maximum((ahi_ref[0] + ahi_ref[1]) / d + b1hi_ref[...], 0.0)
    z_ref[...] = (jnp.dot(hl, w2lo_ref[...], preferred_element_type=F32)
                  + jnp.dot(hh, w2hi_ref[...], preferred_element_type=F32))


def _node_stage2(alo_p, ahi_p, den_p, b1lo, b1hi, W2lo, W2hi):
    rb = 2048
    grid = NPAD // rb
    return pl.pallas_call(
        _node2_body,
        grid=(grid,),
        in_specs=[
            pl.BlockSpec((NC, rb, CH), lambda i: (0, i, 0)),
            pl.BlockSpec((NC, rb, CH), lambda i: (0, i, 0)),
            pl.BlockSpec((NC, rb, 1), lambda i: (0, i, 0)),
            pl.BlockSpec((1, CH), lambda i: (0, 0)),
            pl.BlockSpec((1, CH), lambda i: (0, 0)),
            pl.BlockSpec((CH, 1), lambda i: (0, 0)),
            pl.BlockSpec((CH, 1), lambda i: (0, 0)),
        ],
        out_specs=pl.BlockSpec((rb, 1), lambda i: (i, 0)),
        out_shape=jax.ShapeDtypeStruct((NPAD, 1), F32),
    )(alo_p, ahi_p, den_p, b1lo, b1hi, W2lo, W2hi)


def _edge2_body(srcr_hbm, dstr_hbm, att2_hbm, z_hbm, zden_hbm,
                num_out, den_out,
                src_t, dst_t, w_t, m_t, att2_t, z_t,
                num_s, den_s, ssc, szero):
    c = lax.axis_index("c")
    s = lax.axis_index("s")
    w_id = c * NS + s

    pltpu.async_copy(zden_hbm, num_s.at[pl.ds(s * RPT, RPT)], szero)
    pltpu.async_copy(zden_hbm, den_s.at[pl.ds(s * RPT, RPT)], szero)
    row0 = w_id * NCHUNKS
    pltpu.sync_copy(srcr_hbm.at[pl.ds(row0, NCHUNKS)], src_t)
    pltpu.sync_copy(dstr_hbm.at[pl.ds(row0, NCHUNKS)], dst_t)
    pltpu.sync_copy(att2_hbm, att2_t)
    pltpu.sync_copy(z_hbm, z_t)
    as2 = att2_t[pl.ds(0, 16)]
    ad2 = att2_t[pl.ds(16, 16)]

    def wk(k, carry):
        for g in range(CHUNK // 16):
            sl = pl.ds(g * 16, 16)
            zsrc = plsc.load_gather(z_t, [src_t[k, sl]])
            zdst = plsc.load_gather(z_t, [dst_t[k, sl]])
            a = as2 * zsrc + ad2 * zdst
            a = jnp.where(a > 0, a, 0.2 * a)
            w = jnp.exp(a)
            w_t[k, sl] = w
            m_t[k, sl] = w * zsrc
        return carry

    lax.fori_loop(0, NCHUNKS, wk, 0)

    pltpu.make_async_copy(zden_hbm, num_s.at[pl.ds(s * RPT, RPT)],
                          szero).wait()
    pltpu.make_async_copy(zden_hbm, den_s.at[pl.ds(s * RPT, RPT)],
                          szero).wait()
    plsc.subcore_barrier()

    KB = 5

    def sblk(blk, carry):
        for i in range(KB):
            j = blk * KB + i
            pltpu.async_copy(m_t.at[j], num_s.at[dst_t.at[j]], ssc, add=True)
            pltpu.async_copy(w_t.at[j], den_s.at[dst_t.at[j]], ssc, add=True)
        for i in range(2 * KB):
            pltpu.make_async_copy(w_t.at[0], den_s.at[dst_t.at[0]],
                                  ssc).wait()
        return carry

    lax.fori_loop(0, NCHUNKS // KB, sblk, 0)
    plsc.subcore_barrier()
    pltpu.sync_copy(num_s.at[pl.ds(s * RPT, RPT)],
                    num_out.at[c, pl.ds(s * RPT, RPT)])
    pltpu.sync_copy(den_s.at[pl.ds(s * RPT, RPT)],
                    den_out.at[c, pl.ds(s * RPT, RPT)])


def _edge_stage2(srcr, dstr, att2, z, zden):
    mesh = plsc.VectorSubcoreMesh(core_axis_name="c", subcore_axis_name="s")
    f = pl.kernel(
        _edge2_body,
        out_type=[
            jax.ShapeDtypeStruct((NC, NPAD), F32),
            jax.ShapeDtypeStruct((NC, NPAD), F32),
        ],
        mesh=mesh,
        scratch_types=[
            pltpu.VMEM((NCHUNKS, CHUNK), jnp.int32),
            pltpu.VMEM((NCHUNKS, CHUNK), jnp.int32),
            pltpu.VMEM((NCHUNKS, CHUNK), F32),
            pltpu.VMEM((NCHUNKS, CHUNK), F32),
            pltpu.VMEM((32,), F32),
            pltpu.VMEM((NPAD,), F32),
            pltpu.VMEM_SHARED((NPAD,), F32),
            pltpu.VMEM_SHARED((NPAD,), F32),
            pltpu.SemaphoreType.DMA,
            pltpu.SemaphoreType.DMA,
        ],
        compiler_params=pltpu.CompilerParams(needs_layout_passes=False, use_tc_tiling_on_sc=False),
    )
    return f(srcr, dstr, att2, z, zden)


def _final_body(nump_ref, denp_ref, b2_ref, o_ref):
    o_ref[...] = ((nump_ref[0] + nump_ref[1])
                  / (denp_ref[0] + denp_ref[1] + 1e-16)) + b2_ref[0, 0]


def _final_stage(num_p, den_p, b2):
    return pl.pallas_call(
        _final_body,
        in_specs=[
            pl.BlockSpec((NC, NPAD // C, C), lambda: (0, 0, 0)),
            pl.BlockSpec((NC, NPAD // C, C), lambda: (0, 0, 0)),
            pl.BlockSpec((1, 1), lambda: (0, 0)),
        ],
        out_specs=pl.BlockSpec((NPAD // C, C), lambda: (0, 0)),
        out_shape=jax.ShapeDtypeStruct((NPAD // C, C), F32),
    )(num_p, den_p, b2)


def kernel(x, edge_index, W1, att_src1, att_dst1, b1, W2, att_src2,
           att_dst2, b2):
    x_pad = jnp.concatenate(
        [x, jnp.zeros((NPAD - N, C), F32)], axis=0)
    src = edge_index[0].astype(jnp.int32).reshape(E // CHUNK, CHUNK)
    dst = edge_index[1].astype(jnp.int32).reshape(E // CHUNK, CHUNK)
    asv = att_src1.reshape(C, 1)
    adv = att_dst1.reshape(C, 1)

    xplo, xphi, asrc, adst = _node_stage1(x_pad, W1, asv, adv)

    zacc = jnp.zeros((RPT, CH), F32)
    zden = jnp.zeros((RPT,), F32)
    alo_p, ahi_p, den_p = _edge_stage1(src, dst, asrc.reshape(NPAD),
                                       adst.reshape(NPAD), xplo, xphi,
                                       zacc, zden)

    b1f = b1.reshape(1, C)
    z = _node_stage2(alo_p, ahi_p, den_p.reshape(NC, NPAD, 1),
                     b1f[:, :CH], b1f[:, CH:], W2[:CH], W2[CH:])

    att2 = jnp.concatenate([jnp.broadcast_to(att_src2.reshape(1), (16,)),
                            jnp.broadcast_to(att_dst2.reshape(1), (16,))])
    num_p, den2_p = _edge_stage2(src, dst, att2, z.reshape(NPAD), zden)

    o = _final_stage(num_p.reshape(NC, NPAD // C, C),
                     den2_p.reshape(NC, NPAD // C, C), b2.reshape(1, 1))
    return o.reshape(NPAD)[:N]
